# baseline (speedup 1.0000x reference)
import functools

import jax
import jax.numpy as jnp
from jax import lax
from jax.experimental import pallas as pl
from jax.experimental.pallas import tpu as pltpu
from jax.experimental.pallas import tpu_sc as plsc

N = 10000
E = 320000
NFEAT = 128
HID = 8
IN_HEAD = 8
OUT_HEAD = 1
NCLASS = 16

NPAD = 10240
LANES = 16
C = 128
NCORES = 2
NSUB = 16
NWORK = NCORES * NSUB
KCH = 84
PERW = KCH * C
EPAD = NWORK * PERW

RW1 = 80
RW2 = 32
AW = 16
NB = 512
NGRID = NPAD // NB
TPR = NPAD // NSUB


def _rep8(s, j):
    idx = lax.broadcasted_iota(jnp.int32, (LANES,), 0) // 8 + 2 * j
    return lax.gather(
        s, idx[:, None],
        lax.GatherDimensionNumbers(
            offset_dims=(), collapsed_slice_dims=(0,), start_index_map=(0,)),
        slice_sizes=(1,),
        mode=lax.GatherScatterMode.PROMISE_IN_BOUNDS)


def _zero_rows(buf, nrows, rw):
    z = jnp.zeros((LANES,), jnp.float32)

    def body(i, _):
        for j in range(rw // LANES):
            buf[i, pl.ds(j * LANES, LANES)] = z
        return 0

    lax.fori_loop(0, nrows, body, 0)


def _sc_edge_pass(rw, edge_fn):
    mesh = plsc.VectorSubcoreMesh(core_axis_name="c", subcore_axis_name="s")

    def body(ep_hbm, tab_hbm, adst_hbm, out0, out1,
             idxs, hr0, hr1, ar0, ar1, acc, tabs,
             semi0, semi1, semi2, semi3, semh0, semh1, sema0, sema1):
        cid = lax.axis_index("c")
        sid = lax.axis_index("s")
        g0 = (cid * NSUB + sid) * KCH
        r0 = sid * TPR
        semi = [semi0, semi1, semi2, semi3]
        hrs = [hr0, hr1]
        ars = [ar0, ar1]
        semh = [semh0, semh1]
        sema = [sema0, sema1]

        sl_t = pl.ds(r0, TPR)
        pltpu.sync_copy(tab_hbm.at[sl_t], tabs.at[sl_t])
        _zero_rows(hr0, C, rw)
        for b in range(TPR // C):
            pltpu.sync_copy(hr0, acc.at[pl.ds(r0 + b * C, C)])
        plsc.subcore_barrier()

        def issue_idx(g, sl):
            pltpu.async_copy(ep_hbm.at[g], idxs.at[sl], semi[sl])

        def wait_idx(sl):
            pltpu.make_async_copy(ep_hbm.at[0], idxs.at[sl], semi[sl]).wait()

        def issue_gath(isl, hsl):
            pltpu.async_copy(tabs.at[idxs.at[isl, 0]], hrs[hsl], semh[hsl])
            pltpu.async_copy(adst_hbm.at[idxs.at[isl, 1]], ars[hsl], sema[hsl])

        def wait_gath(isl, hsl):
            pltpu.make_async_copy(
                tabs.at[idxs.at[isl, 0]], hrs[hsl], semh[hsl]).wait()
            pltpu.make_async_copy(
                adst_hbm.at[idxs.at[isl, 1]], ars[hsl], sema[hsl]).wait()

        for p in range(3):
            issue_idx(g0 + p, p)
        wait_idx(0)
        issue_gath(0, 0)

        def outer(kk, _):
            for b in range(4):
                k = 4 * kk + b

                @pl.when(k + 3 < KCH)
                def _():
                    issue_idx(g0 + k + 3, (b + 3) % 4)

                @pl.when(k + 1 < KCH)
                def _():
                    wait_idx((b + 1) % 4)
                    issue_gath((b + 1) % 4, (b + 1) % 2)

                hsl = b % 2
                wait_gath(b, hsl)

                @plsc.parallel_loop(0, C, 1, unroll=4)
                def _(i):
                    edge_fn(i, hrs[hsl], ars[hsl])

                pltpu.sync_copy(hrs[hsl], acc.at[idxs.at[b, 1]], add=True)
            return 0

        lax.fori_loop(0, KCH // 4, outer, 0)
        plsc.subcore_barrier()

        for b in range(TPR // C):
            sl = pl.ds(r0 + b * C, C)
            pltpu.sync_copy(acc.at[sl], hr0)

            @pl.when(cid == 0)
            def _():
                pltpu.sync_copy(hr0, out0.at[sl])

            @pl.when(cid == 1)
            def _():
                pltpu.sync_copy(hr0, out1.at[sl])

    out = jax.ShapeDtypeStruct((NPAD, rw), jnp.float32)
    return pl.kernel(
        body,
        out_type=(out, out),
        mesh=mesh,
        compiler_params=pltpu.CompilerParams(
            use_tc_tiling_on_sc=False, needs_layout_passes=False),
        scratch_types=[
            pltpu.VMEM((4, 2, C), jnp.int32),
            pltpu.VMEM((C, rw), jnp.float32),
            pltpu.VMEM((C, rw), jnp.float32),
            pltpu.VMEM((C, AW), jnp.float32),
            pltpu.VMEM((C, AW), jnp.float32),
            pltpu.VMEM_SHARED((NPAD, rw), jnp.float32),
            pltpu.VMEM_SHARED((NPAD, rw), jnp.float32),
        ] + [pltpu.SemaphoreType.DMA] * 8,
    )


def _edge_fn1(i, hrows, arows):
    a_s = hrows[i, pl.ds(64, LANES)]
    a_d = arows[i, pl.ds(0, LANES)]
    e = a_s + a_d
    e = jnp.where(e > 0, e, 0.2 * e)
    s = jnp.exp(e)
    for j in range(4):
        seg = hrows[i, pl.ds(j * LANES, LANES)]
        hrows[i, pl.ds(j * LANES, LANES)] = seg * _rep8(s, j)
    hrows[i, pl.ds(64, LANES)] = s


def _edge_fn2(i, hrows, arows):
    a_s = hrows[i, pl.ds(LANES, LANES)]
    a_d = arows[i, pl.ds(0, LANES)]
    e = a_s + a_d
    e = jnp.where(e > 0, e, 0.2 * e)
    s = jnp.exp(e)
    hrows[i, pl.ds(0, LANES)] = hrows[i, pl.ds(0, LANES)] * s
    lane0 = (lax.broadcasted_iota(jnp.int32, (LANES,), 0) == 0)
    hrows[i, pl.ds(LANES, LANES)] = jnp.where(lane0, s, 0.0)


def _tc1_body(x_ref, w1_ref, aa_ref, hx_ref, ad_ref):
    h = jnp.dot(x_ref[:], w1_ref[:], preferred_element_type=jnp.float32)
    aa = jnp.dot(h, aa_ref[:], preferred_element_type=jnp.float32)
    z8 = jnp.zeros((NB, 8), jnp.float32)
    hx_ref[:] = jnp.concatenate([h, aa[:, :8], z8], axis=1)
    ad_ref[:] = jnp.concatenate([aa[:, 8:], z8], axis=1)


def _tc2_body(p0_ref, p1_ref, rmat_ref, b1_ref, w2_ref, a2_ref,
              hx_ref, ad_ref):
    u = p0_ref[:] + p1_ref[:]
    rec = 1.0 / (u[:, 64:72] + 1e-16)
    recrep = jnp.dot(rec, rmat_ref[:], preferred_element_type=jnp.float32)
    o1 = u[:, :64] * recrep + b1_ref[:]
    act = jnp.where(o1 > 0, o1, jnp.exp(jnp.minimum(o1, 0.0)) - 1.0)
    h2 = jnp.dot(act, w2_ref[:], preferred_element_type=jnp.float32)
    aa2 = jnp.dot(h2, a2_ref[:], preferred_element_type=jnp.float32)
    hx_ref[:] = jnp.concatenate(
        [h2, jnp.broadcast_to(aa2[:, :1], (NB, LANES))], axis=1)
    ad_ref[:] = jnp.broadcast_to(aa2[:, 1:2], (NB, AW))


def _tc3_body(q0_ref, q1_ref, b2_ref, o_ref):
    u = q0_ref[:] + q1_ref[:]
    den = u[:, 16:17] + 1e-16
    o_ref[:] = u[:, :16] / den + b2_ref[:]


def _full(shape):
    return pl.BlockSpec(shape, lambda i: (0, 0))


def _blk(width):
    return pl.BlockSpec((NB, width), lambda i: (i, 0))


def kernel(x, edge_index, W1, att_src1, att_dst1, bias1,
           W2, att_src2, att_dst2, bias2):
    f32 = jnp.float32

    loop = jnp.arange(N, dtype=jnp.int32)
    padn = jnp.full((EPAD - E - N,), N, jnp.int32)
    src = jnp.concatenate([edge_index[0], loop, padn])
    dst = jnp.concatenate([edge_index[1], loop, padn])
    nch = EPAD // C
    ep = jnp.stack([src.reshape(nch, C), dst.reshape(nch, C)], axis=1)

    xpad = jnp.zeros((NPAD, NFEAT), f32).at[:N].set(x)

    a1s = att_src1.reshape(IN_HEAD, HID)
    a1d = att_dst1.reshape(IN_HEAD, HID)
    eye8 = jnp.eye(IN_HEAD, dtype=f32)
    asrc_m = (eye8[:, None, :] * a1s[:, :, None]).reshape(64, IN_HEAD)
    adst_m = (eye8[:, None, :] * a1d[:, :, None]).reshape(64, IN_HEAD)
    aa1 = jnp.concatenate([asrc_m, adst_m], axis=1)
    rmat = (eye8[:, None, :] * jnp.ones((IN_HEAD, HID, 1), f32)
            ).reshape(64, IN_HEAD).T
    a2 = jnp.stack([att_src2.reshape(NCLASS), att_dst2.reshape(NCLASS)],
                   axis=1)

    tab1, adt1 = pl.pallas_call(
        _tc1_body,
        grid=(NGRID,),
        in_specs=[_blk(NFEAT), _full((NFEAT, 64)), _full((64, 16))],
        out_specs=[_blk(RW1), _blk(AW)],
        out_shape=[jax.ShapeDtypeStruct((NPAD, RW1), f32),
                   jax.ShapeDtypeStruct((NPAD, AW), f32)],
    )(xpad, W1, aa1)

    p0, p1 = _sc_edge_pass(RW1, _edge_fn1)(ep, tab1, adt1)

    tab2, adt2 = pl.pallas_call(
        _tc2_body,
        grid=(NGRID,),
        in_specs=[_blk(RW1), _blk(RW1), _full((IN_HEAD, 64)),
                  _full((1, 64)), _full((64, NCLASS)), _full((NCLASS, 2))],
        out_specs=[_blk(RW2), _blk(AW)],
        out_shape=[jax.ShapeDtypeStruct((NPAD, RW2), f32),
                   jax.ShapeDtypeStruct((NPAD, AW), f32)],
    )(p0, p1, rmat, bias1.reshape(1, 64), W2, a2)

    q0, q1 = _sc_edge_pass(RW2, _edge_fn2)(ep, tab2, adt2)

    out = pl.pallas_call(
        _tc3_body,
        grid=(NGRID,),
        in_specs=[_blk(RW2), _blk(RW2), _full((1, NCLASS))],
        out_specs=_blk(NCLASS),
        out_shape=jax.ShapeDtypeStruct((NPAD, NCLASS), f32),
    )(q0, q1, bias2.reshape(1, NCLASS))

    return out[:N]


# speedup vs baseline: 1.1565x; 1.1565x over previous
import functools

import jax
import jax.numpy as jnp
from jax import lax
from jax.experimental import pallas as pl
from jax.experimental.pallas import tpu as pltpu
from jax.experimental.pallas import tpu_sc as plsc

N = 10000
E = 320000
NFEAT = 128
HID = 8
IN_HEAD = 8
OUT_HEAD = 1
NCLASS = 16

NPAD = 10240
LANES = 16
C = 128
NCORES = 2
NSUB = 16
NWORK = NCORES * NSUB
KCH = 84
PERW = KCH * C
EPAD = NWORK * PERW

RW1 = 72
RW2 = 32
AW = 16
NB = 512
NGRID = NPAD // NB
TPR = NPAD // NSUB


def _rep8(s, j, base=0):
    idx = lax.broadcasted_iota(jnp.int32, (LANES,), 0) // 8 + 2 * j + base
    return lax.gather(
        s, idx[:, None],
        lax.GatherDimensionNumbers(
            offset_dims=(), collapsed_slice_dims=(0,), start_index_map=(0,)),
        slice_sizes=(1,),
        mode=lax.GatherScatterMode.PROMISE_IN_BOUNDS)


def _zero_rows(buf, nrows, rw):
    z = jnp.zeros((LANES,), jnp.float32)
    offs = list(range(0, rw - LANES + 1, LANES))
    if rw % LANES:
        offs.append(rw - LANES)

    def body(i, _):
        for o in offs:
            buf[i, pl.ds(o, LANES)] = z
        return 0

    lax.fori_loop(0, nrows, body, 0)


def _sc_edge_pass(rw, edge_fn):
    mesh = plsc.VectorSubcoreMesh(core_axis_name="c", subcore_axis_name="s")

    def body(ep_hbm, tab_hbm, adst_hbm, out0, out1,
             idxs, hr0, hr1, ar0, ar1, acc, tabs, ads,
             semi0, semi1, semi2, semi3, semh0, semh1, sema0, sema1):
        cid = lax.axis_index("c")
        sid = lax.axis_index("s")
        g0 = (cid * NSUB + sid) * KCH
        r0 = sid * TPR
        semi = [semi0, semi1, semi2, semi3]
        hrs = [hr0, hr1]
        ars = [ar0, ar1]
        semh = [semh0, semh1]
        sema = [sema0, sema1]

        sl_t = pl.ds(r0, TPR)
        pltpu.sync_copy(tab_hbm.at[sl_t], tabs.at[sl_t])
        pltpu.sync_copy(adst_hbm.at[sl_t], ads.at[sl_t])
        _zero_rows(hr0, C, rw)
        for b in range(TPR // C):
            pltpu.sync_copy(hr0, acc.at[pl.ds(r0 + b * C, C)])
        plsc.subcore_barrier()

        def issue_idx(g, sl):
            pltpu.async_copy(ep_hbm.at[g], idxs.at[sl], semi[sl])

        def wait_idx(sl):
            pltpu.make_async_copy(ep_hbm.at[0], idxs.at[sl], semi[sl]).wait()

        def issue_gath(isl, hsl):
            pltpu.async_copy(tabs.at[idxs.at[isl, 0]], hrs[hsl], semh[hsl])
            pltpu.async_copy(ads.at[idxs.at[isl, 1]], ars[hsl], sema[hsl])

        def wait_gath(isl, hsl):
            pltpu.make_async_copy(
                tabs.at[idxs.at[isl, 0]], hrs[hsl], semh[hsl]).wait()
            pltpu.make_async_copy(
                ads.at[idxs.at[isl, 1]], ars[hsl], sema[hsl]).wait()

        for p in range(3):
            issue_idx(g0 + p, p)
        wait_idx(0)
        issue_gath(0, 0)

        def outer(kk, _):
            for b in range(4):
                k = 4 * kk + b

                @pl.when(k + 3 < KCH)
                def _():
                    issue_idx(g0 + k + 3, (b + 3) % 4)

                @pl.when(k + 1 < KCH)
                def _():
                    wait_idx((b + 1) % 4)
                    issue_gath((b + 1) % 4, (b + 1) % 2)

                hsl = b % 2
                wait_gath(b, hsl)

                @plsc.parallel_loop(0, C, 1, unroll=4)
                def _(i):
                    edge_fn(i, hrs[hsl], ars[hsl])

                pltpu.sync_copy(hrs[hsl], acc.at[idxs.at[b, 1]], add=True)
            return 0

        lax.fori_loop(0, KCH // 4, outer, 0)
        plsc.subcore_barrier()

        for b in range(TPR // C):
            sl = pl.ds(r0 + b * C, C)
            pltpu.sync_copy(acc.at[sl], hr0)

            @pl.when(cid == 0)
            def _():
                pltpu.sync_copy(hr0, out0.at[sl])

            @pl.when(cid == 1)
            def _():
                pltpu.sync_copy(hr0, out1.at[sl])

    out = jax.ShapeDtypeStruct((NPAD, rw), jnp.float32)
    return pl.kernel(
        body,
        out_type=(out, out),
        mesh=mesh,
        compiler_params=pltpu.CompilerParams(
            use_tc_tiling_on_sc=False, needs_layout_passes=False),
        scratch_types=[
            pltpu.VMEM((4, 2, C), jnp.int32),
            pltpu.VMEM((C, rw), jnp.float32),
            pltpu.VMEM((C, rw), jnp.float32),
            pltpu.VMEM((C, AW), jnp.float32),
            pltpu.VMEM((C, AW), jnp.float32),
            pltpu.VMEM_SHARED((NPAD, rw), jnp.float32),
            pltpu.VMEM_SHARED((NPAD, rw), jnp.float32),
            pltpu.VMEM_SHARED((NPAD, AW), jnp.float32),
        ] + [pltpu.SemaphoreType.DMA] * 8,
    )


def _edge_fn1(i, hrows, arows):
    iota = lax.broadcasted_iota(jnp.int32, (LANES,), 0)
    a_s = hrows[i, pl.ds(56, LANES)]
    a_d = arows[i, pl.ds(0, LANES)]
    e = a_s + a_d
    e = jnp.where(e > 0, e, 0.2 * e)
    s = jnp.exp(e)
    for j in range(4):
        seg = hrows[i, pl.ds(j * LANES, LANES)]
        hrows[i, pl.ds(j * LANES, LANES)] = seg * _rep8(s, j, base=8)
    row = jnp.zeros((LANES,), jnp.int32) + i
    plsc.store_scatter(hrows, [row, iota + 56], s, mask=iota >= 8)


def _edge_fn2(i, hrows, arows):
    a_s = hrows[i, pl.ds(LANES, LANES)]
    a_d = arows[i, pl.ds(0, LANES)]
    e = a_s + a_d
    e = jnp.where(e > 0, e, 0.2 * e)
    s = jnp.exp(e)
    hrows[i, pl.ds(0, LANES)] = hrows[i, pl.ds(0, LANES)] * s
    lane0 = (lax.broadcasted_iota(jnp.int32, (LANES,), 0) == 0)
    hrows[i, pl.ds(LANES, LANES)] = jnp.where(lane0, s, 0.0)


def _tc1_body(x_ref, w1_ref, aa_ref, hx_ref, ad_ref):
    h = jnp.dot(x_ref[:], w1_ref[:], preferred_element_type=jnp.float32)
    aa = jnp.dot(h, aa_ref[:], preferred_element_type=jnp.float32)
    z8 = jnp.zeros((NB, 8), jnp.float32)
    hx_ref[:] = jnp.concatenate([h, aa[:, :8]], axis=1)
    ad_ref[:] = jnp.concatenate([z8, aa[:, 8:]], axis=1)


def _tc2_body(p0_ref, p1_ref, rmat_ref, b1_ref, w2_ref, a2_ref,
              hx_ref, ad_ref):
    u = p0_ref[:] + p1_ref[:]
    rec = 1.0 / (u[:, 64:72] + 1e-16)
    recrep = jnp.dot(rec, rmat_ref[:], preferred_element_type=jnp.float32)
    o1 = u[:, :64] * recrep + b1_ref[:]
    act = jnp.where(o1 > 0, o1, jnp.exp(jnp.minimum(o1, 0.0)) - 1.0)
    h2 = jnp.dot(act, w2_ref[:], preferred_element_type=jnp.float32)
    aa2 = jnp.dot(h2, a2_ref[:], preferred_element_type=jnp.float32)
    hx_ref[:] = jnp.concatenate(
        [h2, jnp.broadcast_to(aa2[:, :1], (NB, LANES))], axis=1)
    ad_ref[:] = jnp.broadcast_to(aa2[:, 1:2], (NB, AW))


def _tc3_body(q0_ref, q1_ref, b2_ref, o_ref):
    u = q0_ref[:] + q1_ref[:]
    den = u[:, 16:17] + 1e-16
    o_ref[:] = u[:, :16] / den + b2_ref[:]


def _full(shape):
    return pl.BlockSpec(shape, lambda i: (0, 0))


def _blk(width):
    return pl.BlockSpec((NB, width), lambda i: (i, 0))


def kernel(x, edge_index, W1, att_src1, att_dst1, bias1,
           W2, att_src2, att_dst2, bias2):
    f32 = jnp.float32

    loop = jnp.arange(N, dtype=jnp.int32)
    padn = jnp.full((EPAD - E - N,), N, jnp.int32)
    src = jnp.concatenate([edge_index[0], loop, padn])
    dst = jnp.concatenate([edge_index[1], loop, padn])
    nch = EPAD // C
    ep = jnp.stack([src.reshape(nch, C), dst.reshape(nch, C)], axis=1)

    xpad = jnp.zeros((NPAD, NFEAT), f32).at[:N].set(x)

    a1s = att_src1.reshape(IN_HEAD, HID)
    a1d = att_dst1.reshape(IN_HEAD, HID)
    eye8 = jnp.eye(IN_HEAD, dtype=f32)
    asrc_m = (eye8[:, None, :] * a1s[:, :, None]).reshape(64, IN_HEAD)
    adst_m = (eye8[:, None, :] * a1d[:, :, None]).reshape(64, IN_HEAD)
    aa1 = jnp.concatenate([asrc_m, adst_m], axis=1)
    rmat = (eye8[:, None, :] * jnp.ones((IN_HEAD, HID, 1), f32)
            ).reshape(64, IN_HEAD).T
    a2 = jnp.stack([att_src2.reshape(NCLASS), att_dst2.reshape(NCLASS)],
                   axis=1)

    tab1, adt1 = pl.pallas_call(
        _tc1_body,
        grid=(NGRID,),
        in_specs=[_blk(NFEAT), _full((NFEAT, 64)), _full((64, 16))],
        out_specs=[_blk(RW1), _blk(AW)],
        out_shape=[jax.ShapeDtypeStruct((NPAD, RW1), f32),
                   jax.ShapeDtypeStruct((NPAD, AW), f32)],
    )(xpad, W1, aa1)

    p0, p1 = _sc_edge_pass(RW1, _edge_fn1)(ep, tab1, adt1)

    tab2, adt2 = pl.pallas_call(
        _tc2_body,
        grid=(NGRID,),
        in_specs=[_blk(RW1), _blk(RW1), _full((IN_HEAD, 64)),
                  _full((1, 64)), _full((64, NCLASS)), _full((NCLASS, 2))],
        out_specs=[_blk(RW2), _blk(AW)],
        out_shape=[jax.ShapeDtypeStruct((NPAD, RW2), f32),
                   jax.ShapeDtypeStruct((NPAD, AW), f32)],
    )(p0, p1, rmat, bias1.reshape(1, 64), W2, a2)

    q0, q1 = _sc_edge_pass(RW2, _edge_fn2)(ep, tab2, adt2)

    out = pl.pallas_call(
        _tc3_body,
        grid=(NGRID,),
        in_specs=[_blk(RW2), _blk(RW2), _full((1, NCLASS))],
        out_specs=_blk(NCLASS),
        out_shape=jax.ShapeDtypeStruct((NPAD, NCLASS), f32),
    )(q0, q1, bias2.reshape(1, NCLASS))

    return out[:N]


# speedup vs baseline: 1.3894x; 1.2014x over previous
import functools

import jax
import jax.numpy as jnp
from jax import lax
from jax.experimental import pallas as pl
from jax.experimental.pallas import tpu as pltpu
from jax.experimental.pallas import tpu_sc as plsc

N = 10000
E = 320000
NFEAT = 128
HID = 8
IN_HEAD = 8
OUT_HEAD = 1
NCLASS = 16

NPAD = 10240
LANES = 16
C = 128
NCORES = 2
NSUB = 16
NWORK = NCORES * NSUB
KCH = 84
PERW = KCH * C
EPAD = NWORK * PERW

RW1 = 72
RW2 = 32
AW = 16
NB = 512
NGRID = NPAD // NB
TPR = NPAD // NSUB


def _rep8(s, j, base=0):
    idx = lax.broadcasted_iota(jnp.int32, (LANES,), 0) // 8 + 2 * j + base
    return lax.gather(
        s, idx[:, None],
        lax.GatherDimensionNumbers(
            offset_dims=(), collapsed_slice_dims=(0,), start_index_map=(0,)),
        slice_sizes=(1,),
        mode=lax.GatherScatterMode.PROMISE_IN_BOUNDS)


def _zero_rows(buf, nrows, rw):
    z = jnp.zeros((LANES,), jnp.float32)
    offs = list(range(0, rw - LANES + 1, LANES))
    if rw % LANES:
        offs.append(rw - LANES)

    def body(i, _):
        for o in offs:
            buf[i, pl.ds(o, LANES)] = z
        return 0

    lax.fori_loop(0, nrows, body, 0)


def _sc_edge_pass(rw, edge_fn):
    mesh = plsc.VectorSubcoreMesh(core_axis_name="c", subcore_axis_name="s")

    def body(ep_hbm, tab_hbm, adst_hbm, out0, out1,
             idxs, hr0, hr1, ar0, ar1, acc, tabs, ads,
             semi0, semi1, semi2, semi3, semh0, semh1, sema0, sema1,
             semc0, semc1):
        cid = lax.axis_index("c")
        sid = lax.axis_index("s")
        w = cid * NSUB + sid
        r0 = sid * TPR
        semi = [semi0, semi1, semi2, semi3]
        hrs = [hr0, hr1]
        ars = [ar0, ar1]
        semh = [semh0, semh1]
        sema = [sema0, sema1]
        semc = [semc0, semc1]

        sl_t = pl.ds(r0, TPR)
        pltpu.sync_copy(tab_hbm.at[sl_t], tabs.at[sl_t])
        pltpu.sync_copy(adst_hbm.at[sl_t], ads.at[sl_t])
        _zero_rows(hr0, C, rw)
        for b in range(TPR // C):
            pltpu.sync_copy(hr0, acc.at[pl.ds(r0 + b * C, C)])
        plsc.subcore_barrier()

        def issue_idx(k, sl):
            pltpu.async_copy(ep_hbm.at[k * NWORK + w], idxs.at[sl], semi[sl])

        def wait_idx(sl):
            pltpu.make_async_copy(ep_hbm.at[0], idxs.at[sl], semi[sl]).wait()

        def issue_gath(isl, hsl):
            pltpu.async_copy(tabs.at[idxs.at[isl, 0]], hrs[hsl], semh[hsl])
            pltpu.async_copy(ads.at[idxs.at[isl, 1]], ars[hsl], sema[hsl])

        def wait_gath(isl, hsl):
            pltpu.make_async_copy(
                tabs.at[idxs.at[isl, 0]], hrs[hsl], semh[hsl]).wait()
            pltpu.make_async_copy(
                ads.at[idxs.at[isl, 1]], ars[hsl], sema[hsl]).wait()

        def wait_scat(hsl):
            pltpu.make_async_copy(
                hrs[hsl], acc.at[idxs.at[0, 1]], semc[hsl]).wait()

        issue_idx(0, 0)
        issue_idx(1, 1)
        wait_idx(0)
        issue_gath(0, 0)

        def outer(kk, _):
            for b in range(4):
                k = 4 * kk + b

                @pl.when(k + 2 < KCH)
                def _():
                    issue_idx(k + 2, (b + 2) % 4)

                @pl.when(jnp.logical_and(k + 1 < KCH, k >= 1))
                def _():
                    wait_scat((b + 1) % 2)

                @pl.when(k + 1 < KCH)
                def _():
                    wait_idx((b + 1) % 4)
                    issue_gath((b + 1) % 4, (b + 1) % 2)

                hsl = b % 2
                wait_gath(b, hsl)

                @plsc.parallel_loop(0, C, 1, unroll=4)
                def _(i):
                    edge_fn(i, hrs[hsl], ars[hsl])

                pltpu.async_copy(
                    hrs[hsl], acc.at[idxs.at[b, 1]], semc[hsl], add=True)
            return 0

        lax.fori_loop(0, KCH // 4, outer, 0)
        wait_scat(0)
        wait_scat(1)
        plsc.subcore_barrier()

        for b in range(TPR // C):
            sl = pl.ds(r0 + b * C, C)
            pltpu.sync_copy(acc.at[sl], hr0)

            @pl.when(cid == 0)
            def _():
                pltpu.sync_copy(hr0, out0.at[sl])

            @pl.when(cid == 1)
            def _():
                pltpu.sync_copy(hr0, out1.at[sl])

    out = jax.ShapeDtypeStruct((NPAD, rw), jnp.float32)
    return pl.kernel(
        body,
        out_type=(out, out),
        mesh=mesh,
        compiler_params=pltpu.CompilerParams(
            use_tc_tiling_on_sc=False, needs_layout_passes=False),
        scratch_types=[
            pltpu.VMEM((4, 2, C), jnp.int32),
            pltpu.VMEM((C, rw), jnp.float32),
            pltpu.VMEM((C, rw), jnp.float32),
            pltpu.VMEM((C, AW), jnp.float32),
            pltpu.VMEM((C, AW), jnp.float32),
            pltpu.VMEM_SHARED((NPAD, rw), jnp.float32),
            pltpu.VMEM_SHARED((NPAD, rw), jnp.float32),
            pltpu.VMEM_SHARED((NPAD, AW), jnp.float32),
        ] + [pltpu.SemaphoreType.DMA] * 10,
    )


def _edge_fn1(i, hrows, arows):
    iota = lax.broadcasted_iota(jnp.int32, (LANES,), 0)
    a_s = hrows[i, pl.ds(56, LANES)]
    a_d = arows[i, pl.ds(0, LANES)]
    e = a_s + a_d
    e = jnp.where(e > 0, e, 0.2 * e)
    s = jnp.exp(e)
    for j in range(4):
        seg = hrows[i, pl.ds(j * LANES, LANES)]
        hrows[i, pl.ds(j * LANES, LANES)] = seg * _rep8(s, j, base=8)
    row = jnp.zeros((LANES,), jnp.int32) + i
    plsc.store_scatter(hrows, [row, iota + 56], s, mask=iota >= 8)


def _edge_fn2(i, hrows, arows):
    a_s = hrows[i, pl.ds(LANES, LANES)]
    a_d = arows[i, pl.ds(0, LANES)]
    e = a_s + a_d
    e = jnp.where(e > 0, e, 0.2 * e)
    s = jnp.exp(e)
    hrows[i, pl.ds(0, LANES)] = hrows[i, pl.ds(0, LANES)] * s
    lane0 = (lax.broadcasted_iota(jnp.int32, (LANES,), 0) == 0)
    hrows[i, pl.ds(LANES, LANES)] = jnp.where(lane0, s, 0.0)


def _tc1_body(x_ref, w1_ref, aa_ref, hx_ref, ad_ref):
    h = jnp.dot(x_ref[:], w1_ref[:], preferred_element_type=jnp.float32)
    aa = jnp.dot(h, aa_ref[:], preferred_element_type=jnp.float32)
    z8 = jnp.zeros((NB, 8), jnp.float32)
    hx_ref[:] = jnp.concatenate([h, aa[:, :8]], axis=1)
    ad_ref[:] = jnp.concatenate([z8, aa[:, 8:]], axis=1)


def _tc2_body(p0_ref, p1_ref, rmat_ref, b1_ref, w2_ref, a2_ref,
              hx_ref, ad_ref):
    u = p0_ref[:] + p1_ref[:]
    rec = 1.0 / (u[:, 64:72] + 1e-16)
    recrep = jnp.dot(rec, rmat_ref[:], preferred_element_type=jnp.float32)
    o1 = u[:, :64] * recrep + b1_ref[:]
    act = jnp.where(o1 > 0, o1, jnp.exp(jnp.minimum(o1, 0.0)) - 1.0)
    h2 = jnp.dot(act, w2_ref[:], preferred_element_type=jnp.float32)
    aa2 = jnp.dot(h2, a2_ref[:], preferred_element_type=jnp.float32)
    hx_ref[:] = jnp.concatenate(
        [h2, jnp.broadcast_to(aa2[:, :1], (NB, LANES))], axis=1)
    ad_ref[:] = jnp.broadcast_to(aa2[:, 1:2], (NB, AW))


def _tc3_body(q0_ref, q1_ref, b2_ref, o_ref):
    u = q0_ref[:] + q1_ref[:]
    den = u[:, 16:17] + 1e-16
    o_ref[:] = u[:, :16] / den + b2_ref[:]


def _full(shape):
    return pl.BlockSpec(shape, lambda i: (0, 0))


def _blk(width):
    return pl.BlockSpec((NB, width), lambda i: (i, 0))


def kernel(x, edge_index, W1, att_src1, att_dst1, bias1,
           W2, att_src2, att_dst2, bias2):
    f32 = jnp.float32

    loop = jnp.arange(N, dtype=jnp.int32)
    padn = N + jnp.arange(EPAD - E - N, dtype=jnp.int32) % (NPAD - N)
    src = jnp.concatenate([edge_index[0], loop, padn])
    dst = jnp.concatenate([edge_index[1], loop, padn])
    nch = EPAD // C
    ep = jnp.stack([src.reshape(nch, C), dst.reshape(nch, C)], axis=1)

    xpad = jnp.zeros((NPAD, NFEAT), f32).at[:N].set(x)

    a1s = att_src1.reshape(IN_HEAD, HID)
    a1d = att_dst1.reshape(IN_HEAD, HID)
    eye8 = jnp.eye(IN_HEAD, dtype=f32)
    asrc_m = (eye8[:, None, :] * a1s[:, :, None]).reshape(64, IN_HEAD)
    adst_m = (eye8[:, None, :] * a1d[:, :, None]).reshape(64, IN_HEAD)
    aa1 = jnp.concatenate([asrc_m, adst_m], axis=1)
    rmat = (eye8[:, None, :] * jnp.ones((IN_HEAD, HID, 1), f32)
            ).reshape(64, IN_HEAD).T
    a2 = jnp.stack([att_src2.reshape(NCLASS), att_dst2.reshape(NCLASS)],
                   axis=1)

    tab1, adt1 = pl.pallas_call(
        _tc1_body,
        grid=(NGRID,),
        in_specs=[_blk(NFEAT), _full((NFEAT, 64)), _full((64, 16))],
        out_specs=[_blk(RW1), _blk(AW)],
        out_shape=[jax.ShapeDtypeStruct((NPAD, RW1), f32),
                   jax.ShapeDtypeStruct((NPAD, AW), f32)],
    )(xpad, W1, aa1)

    p0, p1 = _sc_edge_pass(RW1, _edge_fn1)(ep, tab1, adt1)

    tab2, adt2 = pl.pallas_call(
        _tc2_body,
        grid=(NGRID,),
        in_specs=[_blk(RW1), _blk(RW1), _full((IN_HEAD, 64)),
                  _full((1, 64)), _full((64, NCLASS)), _full((NCLASS, 2))],
        out_specs=[_blk(RW2), _blk(AW)],
        out_shape=[jax.ShapeDtypeStruct((NPAD, RW2), f32),
                   jax.ShapeDtypeStruct((NPAD, AW), f32)],
    )(p0, p1, rmat, bias1.reshape(1, 64), W2, a2)

    q0, q1 = _sc_edge_pass(RW2, _edge_fn2)(ep, tab2, adt2)

    out = pl.pallas_call(
        _tc3_body,
        grid=(NGRID,),
        in_specs=[_blk(RW2), _blk(RW2), _full((1, NCLASS))],
        out_specs=_blk(NCLASS),
        out_shape=jax.ShapeDtypeStruct((NPAD, NCLASS), f32),
    )(q0, q1, bias2.reshape(1, NCLASS))

    return out[:N]
